# baseline (speedup 1.0000x reference)
import functools

import jax
import jax.numpy as jnp
from jax import lax
from jax.experimental import pallas as pl
from jax.experimental.pallas import tpu as pltpu
from jax.experimental.pallas import tpu_sc as plsc

NUM_CLASSES = 100000
EMB_DIM = 128
BATCH = 16384

_NC = 2
_NS = 16
_NW = _NC * _NS
_BPW = BATCH // _NW


def _make_sc_gather():
    mesh = plsc.VectorSubcoreMesh(core_axis_name="c", subcore_axis_name="s")

    @functools.partial(
        pl.kernel,
        mesh=mesh,
        out_type=jax.ShapeDtypeStruct((BATCH, EMB_DIM), jnp.float32),
        scratch_types=[
            pltpu.VMEM((_BPW,), jnp.int32),
            pltpu.VMEM((_BPW, EMB_DIM), jnp.float32),
            pltpu.SemaphoreType.DMA,
        ],
    )
    def gather_k(labels_hbm, table_hbm, out_hbm, idx_v, rows_v, sem):
        wid = lax.axis_index("s") * _NC + lax.axis_index("c")
        base = wid * _BPW
        pltpu.sync_copy(labels_hbm.at[pl.ds(base, _BPW)], idx_v)
        pltpu.async_copy(table_hbm.at[idx_v], rows_v, sem).wait()
        pltpu.sync_copy(rows_v, out_hbm.at[pl.ds(base, _BPW)])

    return gather_k


_sc_gather = _make_sc_gather()

_NSTRIPE = 4
_SROWS = BATCH // _NSTRIPE


def _silu_linear(x_hbm, w_ref, b_ref, o_hbm, x_v, y_v, in_sems, out_sems):
    for i in range(_NSTRIPE):
        pltpu.make_async_copy(
            x_hbm.at[pl.ds(i * _SROWS, _SROWS)], x_v.at[i], in_sems.at[i]
        ).start()
    for i in range(_NSTRIPE):
        pltpu.make_async_copy(
            x_hbm.at[pl.ds(i * _SROWS, _SROWS)], x_v.at[i], in_sems.at[i]
        ).wait()
        x = x_v[i]
        h = x * jax.nn.sigmoid(x)
        y_v[i] = (
            lax.dot_general(
                h, w_ref[...], (((1,), (1,)), ((), ())),
                preferred_element_type=jnp.float32,
            )
            + b_ref[...]
        )
        pltpu.make_async_copy(
            y_v.at[i], o_hbm.at[pl.ds(i * _SROWS, _SROWS)], out_sems.at[i]
        ).start()
    for i in range(_NSTRIPE):
        pltpu.make_async_copy(
            y_v.at[i], o_hbm.at[pl.ds(i * _SROWS, _SROWS)], out_sems.at[i]
        ).wait()


def kernel(labels, table, W, b):
    labels = labels.astype(jnp.int32)
    b2 = b.reshape(1, EMB_DIM)
    gathered = _sc_gather(labels, table)
    out = pl.pallas_call(
        _silu_linear,
        in_specs=[
            pl.BlockSpec(memory_space=pl.ANY),
            pl.BlockSpec((EMB_DIM, EMB_DIM), lambda: (0, 0)),
            pl.BlockSpec((1, EMB_DIM), lambda: (0, 0)),
        ],
        out_specs=pl.BlockSpec(memory_space=pl.ANY),
        out_shape=jax.ShapeDtypeStruct((BATCH, EMB_DIM), jnp.float32),
        scratch_shapes=[
            pltpu.VMEM((_NSTRIPE, _SROWS, EMB_DIM), jnp.float32),
            pltpu.VMEM((_NSTRIPE, _SROWS, EMB_DIM), jnp.float32),
            pltpu.SemaphoreType.DMA((_NSTRIPE,)),
            pltpu.SemaphoreType.DMA((_NSTRIPE,)),
        ],
    )(gathered, W, b2)
    return out


# speedup vs baseline: 1.0016x; 1.0016x over previous
import functools

import jax
import jax.numpy as jnp
from jax import lax
from jax.experimental import pallas as pl
from jax.experimental.pallas import tpu as pltpu
from jax.experimental.pallas import tpu_sc as plsc

NUM_CLASSES = 100000
EMB_DIM = 128
BATCH = 16384

_NC = 2
_NS = 16
_NW = _NC * _NS
_BPW = BATCH // _NW
_NSUB = 4
_SUB = _BPW // _NSUB


def _make_sc_gather():
    mesh = plsc.VectorSubcoreMesh(core_axis_name="c", subcore_axis_name="s")

    @functools.partial(
        pl.kernel,
        mesh=mesh,
        out_type=jax.ShapeDtypeStruct((BATCH, EMB_DIM), jnp.float32),
        scratch_types=[
            pltpu.VMEM((_NSUB, _SUB), jnp.int32),
            pltpu.VMEM((_NSUB, _SUB, EMB_DIM), jnp.float32),
            pltpu.SemaphoreType.DMA((_NSUB,)),
            pltpu.SemaphoreType.DMA((_NSUB,)),
            pltpu.SemaphoreType.DMA((_NSUB,)),
        ],
    )
    def gather_k(labels_hbm, table_hbm, out_hbm, idx_v, rows_v,
                 isem, gsem, wsem):
        wid = lax.axis_index("s") * _NC + lax.axis_index("c")
        base = wid * _BPW
        idx_cps = []
        for i in range(_NSUB):
            cp = pltpu.make_async_copy(
                labels_hbm.at[pl.ds(base + i * _SUB, _SUB)],
                idx_v.at[i],
                isem.at[i],
            )
            cp.start()
            idx_cps.append(cp)
        g_cps = []
        for i in range(_NSUB):
            idx_cps[i].wait()
            cp = pltpu.make_async_copy(
                table_hbm.at[idx_v.at[i]], rows_v.at[i], gsem.at[i]
            )
            cp.start()
            g_cps.append(cp)
        w_cps = []
        for i in range(_NSUB):
            g_cps[i].wait()
            cp = pltpu.make_async_copy(
                rows_v.at[i],
                out_hbm.at[pl.ds(base + i * _SUB, _SUB)],
                wsem.at[i],
            )
            cp.start()
            w_cps.append(cp)
        for i in range(_NSUB):
            w_cps[i].wait()

    return gather_k


_sc_gather = _make_sc_gather()

_BLK = 8192


def _silu_linear(x_ref, w_ref, b_ref, o_ref):
    x = x_ref[...]
    h = x * jax.nn.sigmoid(x)
    o_ref[...] = (
        lax.dot_general(
            h, w_ref[...], (((1,), (1,)), ((), ())),
            preferred_element_type=jnp.float32,
        )
        + b_ref[...]
    )


def kernel(labels, table, W, b):
    labels = labels.astype(jnp.int32)
    b2 = b.reshape(1, EMB_DIM)
    gathered = _sc_gather(labels, table)
    out = pl.pallas_call(
        _silu_linear,
        grid=(BATCH // _BLK,),
        in_specs=[
            pl.BlockSpec((_BLK, EMB_DIM), lambda i: (i, 0)),
            pl.BlockSpec((EMB_DIM, EMB_DIM), lambda i: (0, 0)),
            pl.BlockSpec((1, EMB_DIM), lambda i: (0, 0)),
        ],
        out_specs=pl.BlockSpec((_BLK, EMB_DIM), lambda i: (i, 0)),
        out_shape=jax.ShapeDtypeStruct((BATCH, EMB_DIM), jnp.float32),
    )(gathered, W, b2)
    return out


# speedup vs baseline: 1.0038x; 1.0022x over previous
import functools

import jax
import jax.numpy as jnp
from jax import lax
from jax.experimental import pallas as pl
from jax.experimental.pallas import tpu as pltpu
from jax.experimental.pallas import tpu_sc as plsc

NUM_CLASSES = 100000
EMB_DIM = 128
BATCH = 16384

_NC = 2
_NS = 16
_NW = _NC * _NS
_BPW = BATCH // _NW


def _make_sc_gather():
    mesh = plsc.VectorSubcoreMesh(core_axis_name="c", subcore_axis_name="s")

    @functools.partial(
        pl.kernel,
        mesh=mesh,
        out_type=jax.ShapeDtypeStruct((BATCH, EMB_DIM), jnp.float32),
        scratch_types=[
            pltpu.VMEM((_BPW,), jnp.int32),
            pltpu.VMEM((_BPW, EMB_DIM), jnp.float32),
            pltpu.SemaphoreType.DMA,
        ],
    )
    def gather_k(labels_hbm, table_hbm, out_hbm, idx_v, rows_v, sem):
        wid = lax.axis_index("s") * _NC + lax.axis_index("c")
        base = wid * _BPW
        pltpu.sync_copy(labels_hbm.at[pl.ds(base, _BPW)], idx_v)
        pltpu.async_copy(table_hbm.at[idx_v], rows_v, sem).wait()
        pltpu.sync_copy(rows_v, out_hbm.at[pl.ds(base, _BPW)])

    return gather_k


_sc_gather = _make_sc_gather()

_BLK = 8192


def _silu_linear(x_ref, w_ref, b_ref, o_ref):
    x = x_ref[...]
    h = x * jax.nn.sigmoid(x)
    o_ref[...] = (
        lax.dot_general(
            h, w_ref[...], (((1,), (1,)), ((), ())),
            preferred_element_type=jnp.float32,
        )
        + b_ref[...]
    )


def kernel(labels, table, W, b):
    labels = labels.astype(jnp.int32)
    b2 = b.reshape(1, EMB_DIM)
    gathered = _sc_gather(labels, table)
    out = pl.pallas_call(
        _silu_linear,
        grid=(BATCH // _BLK,),
        in_specs=[
            pl.BlockSpec((_BLK, EMB_DIM), lambda i: (i, 0)),
            pl.BlockSpec((EMB_DIM, EMB_DIM), lambda i: (0, 0)),
            pl.BlockSpec((1, EMB_DIM), lambda i: (0, 0)),
        ],
        out_specs=pl.BlockSpec((_BLK, EMB_DIM), lambda i: (i, 0)),
        out_shape=jax.ShapeDtypeStruct((BATCH, EMB_DIM), jnp.float32),
    )(gathered, W, b2)
    return out
